# Initial kernel scaffold; baseline (speedup 1.0000x reference)
import jax
import jax.numpy as jnp
from jax.experimental import pallas as pl


def kernel(embedding, coordinates, batch_index, natoms, total_charge, W_wi, b_wi, W_qi, b_qi):
    raise NotImplementedError("write your pallas kernel here")


# speedup vs baseline: 3.8091x; 3.8091x over previous
import functools

import jax
import jax.numpy as jnp
from jax.experimental import pallas as pl

N = 32768
D = 512
C = 10
S = 16
BN = 2048
GRID = N // BN


def _phase_a(emb_ref, bi_ref, wwi_ref, bwi_ref, wqi_ref, bqi_ref,
             wi_ref, qt_ref, sums_ref):
    emb = emb_ref[...]
    h_w = jnp.dot(emb, wwi_ref[...], preferred_element_type=jnp.float32)
    h_q = jnp.dot(emb, wqi_ref[...], preferred_element_type=jnp.float32)
    wi = jax.nn.softplus(h_w + bwi_ref[...])
    qt = h_q + bqi_ref[...]
    wi_ref[...] = wi
    qt_ref[...] = qt

    bi = bi_ref[...]
    oh = (bi == jax.lax.broadcasted_iota(jnp.int32, (BN, S), 1)
          ).astype(jnp.float32)
    part_w = jax.lax.dot_general(
        oh, wi, (((0,), (0,)), ((), ())),
        preferred_element_type=jnp.float32)
    part_q = jax.lax.dot_general(
        oh, qt, (((0,), (0,)), ((), ())),
        preferred_element_type=jnp.float32)
    part = jnp.concatenate([part_w, part_q], axis=0)

    @pl.when(pl.program_id(0) == 0)
    def _init():
        sums_ref[...] = part

    @pl.when(pl.program_id(0) != 0)
    def _acc():
        sums_ref[...] += part


def _phase_b(wi_ref, qt_ref, bi_ref, sums_ref, qtot_ref, q_ref):
    sums = sums_ref[...]
    wsum = sums[:S, :]
    qsum = sums[S:, :]
    dq = qtot_ref[...] - qsum
    fsys = jnp.where(wsum > 0, dq / jnp.where(wsum > 0, wsum, 1.0), 0.0)
    bi = bi_ref[...]
    oh = (bi == jax.lax.broadcasted_iota(jnp.int32, (BN, S), 1)
          ).astype(jnp.float32)
    f = jnp.dot(oh, fsys, preferred_element_type=jnp.float32)
    q_ref[...] = qt_ref[...] + wi_ref[...] * f


@jax.jit
def _run(embedding, batch_index, total_charge, W_wi, b_wi, W_qi, b_qi):
    bi2 = batch_index.reshape(N, 1)
    bwi = b_wi.reshape(1, C)
    bqi = b_qi.reshape(1, C)
    qtot = total_charge.reshape(S, 1)

    wi, qt, sums = pl.pallas_call(
        _phase_a,
        grid=(GRID,),
        in_specs=[
            pl.BlockSpec((BN, D), lambda i: (i, 0)),
            pl.BlockSpec((BN, 1), lambda i: (i, 0)),
            pl.BlockSpec((D, C), lambda i: (0, 0)),
            pl.BlockSpec((1, C), lambda i: (0, 0)),
            pl.BlockSpec((D, C), lambda i: (0, 0)),
            pl.BlockSpec((1, C), lambda i: (0, 0)),
        ],
        out_specs=[
            pl.BlockSpec((BN, C), lambda i: (i, 0)),
            pl.BlockSpec((BN, C), lambda i: (i, 0)),
            pl.BlockSpec((2 * S, C), lambda i: (0, 0)),
        ],
        out_shape=[
            jax.ShapeDtypeStruct((N, C), jnp.float32),
            jax.ShapeDtypeStruct((N, C), jnp.float32),
            jax.ShapeDtypeStruct((2 * S, C), jnp.float32),
        ],
    )(embedding, bi2, W_wi, bwi, W_qi, bqi)

    q = pl.pallas_call(
        _phase_b,
        grid=(GRID,),
        in_specs=[
            pl.BlockSpec((BN, C), lambda i: (i, 0)),
            pl.BlockSpec((BN, C), lambda i: (i, 0)),
            pl.BlockSpec((BN, 1), lambda i: (i, 0)),
            pl.BlockSpec((2 * S, C), lambda i: (0, 0)),
            pl.BlockSpec((S, 1), lambda i: (0, 0)),
        ],
        out_specs=pl.BlockSpec((BN, C), lambda i: (i, 0)),
        out_shape=jax.ShapeDtypeStruct((N, C), jnp.float32),
    )(wi, qt, bi2, sums, qtot)
    return q


def kernel(embedding, coordinates, batch_index, natoms, total_charge,
           W_wi, b_wi, W_qi, b_qi):
    del coordinates, natoms
    return _run(embedding.astype(jnp.float32), batch_index,
                total_charge.astype(jnp.float32), W_wi, b_wi, W_qi, b_qi)


# speedup vs baseline: 4.1837x; 1.0984x over previous
import jax
import jax.numpy as jnp
from jax.experimental import pallas as pl

N = 32768
D = 512
C = 10
S = 16
BN = 2048
GRID = N // BN


def _phase_a(emb_ref, bi_ref, w_ref, b_ref, hact_ref, sums_ref):
    emb = emb_ref[...]
    h = jnp.dot(emb, w_ref[...],
                preferred_element_type=jnp.float32) + b_ref[...]
    lane = jax.lax.broadcasted_iota(jnp.int32, (BN, 2 * C), 1)
    hact = jnp.where(lane < C, jax.nn.softplus(h), h)
    hact_ref[...] = hact

    bi = bi_ref[...]
    oh = (bi == jax.lax.broadcasted_iota(jnp.int32, (BN, S), 1)
          ).astype(jnp.float32)
    part = jax.lax.dot_general(
        oh, hact, (((0,), (0,)), ((), ())),
        preferred_element_type=jnp.float32)

    @pl.when(pl.program_id(0) == 0)
    def _init():
        sums_ref[...] = part

    @pl.when(pl.program_id(0) != 0)
    def _acc():
        sums_ref[...] += part


def _phase_b(hact_ref, bi_ref, sums_ref, qtot_ref, q_ref):
    sums = sums_ref[...]
    wsum = sums[:, :C]
    qsum = sums[:, C:]
    dq = qtot_ref[...] - qsum
    fsys = jnp.where(wsum > 0, dq / jnp.where(wsum > 0, wsum, 1.0), 0.0)
    bi = bi_ref[...]
    oh = (bi == jax.lax.broadcasted_iota(jnp.int32, (BN, S), 1)
          ).astype(jnp.float32)
    f = jnp.dot(oh, fsys, preferred_element_type=jnp.float32)
    hact = hact_ref[...]
    q_ref[...] = hact[:, C:] + hact[:, :C] * f


@jax.jit
def _run(embedding, batch_index, total_charge, W_wi, b_wi, W_qi, b_qi):
    bi2 = batch_index.reshape(N, 1)
    w_cat = jnp.concatenate([W_wi, W_qi], axis=1)
    b_cat = jnp.concatenate([b_wi, b_qi]).reshape(1, 2 * C)
    qtot = total_charge.reshape(S, 1)

    hact, sums = pl.pallas_call(
        _phase_a,
        grid=(GRID,),
        in_specs=[
            pl.BlockSpec((BN, D), lambda i: (i, 0)),
            pl.BlockSpec((BN, 1), lambda i: (i, 0)),
            pl.BlockSpec((D, 2 * C), lambda i: (0, 0)),
            pl.BlockSpec((1, 2 * C), lambda i: (0, 0)),
        ],
        out_specs=[
            pl.BlockSpec((BN, 2 * C), lambda i: (i, 0)),
            pl.BlockSpec((S, 2 * C), lambda i: (0, 0)),
        ],
        out_shape=[
            jax.ShapeDtypeStruct((N, 2 * C), jnp.float32),
            jax.ShapeDtypeStruct((S, 2 * C), jnp.float32),
        ],
    )(embedding, bi2, w_cat, b_cat)

    q = pl.pallas_call(
        _phase_b,
        grid=(GRID,),
        in_specs=[
            pl.BlockSpec((BN, 2 * C), lambda i: (i, 0)),
            pl.BlockSpec((BN, 1), lambda i: (i, 0)),
            pl.BlockSpec((S, 2 * C), lambda i: (0, 0)),
            pl.BlockSpec((S, 1), lambda i: (0, 0)),
        ],
        out_specs=pl.BlockSpec((BN, C), lambda i: (i, 0)),
        out_shape=jax.ShapeDtypeStruct((N, C), jnp.float32),
    )(hact, bi2, sums, qtot)
    return q


def kernel(embedding, coordinates, batch_index, natoms, total_charge,
           W_wi, b_wi, W_qi, b_qi):
    del coordinates, natoms
    return _run(embedding.astype(jnp.float32), batch_index,
                total_charge.astype(jnp.float32), W_wi, b_wi, W_qi, b_qi)
